# baseline (speedup 1.0000x reference)
import functools

import jax
import jax.numpy as jnp
from jax import lax
from jax.experimental import pallas as pl
from jax.experimental.pallas import tpu as pltpu
from jax.experimental.pallas import tpu_sc as plsc

NC = 2
NS = 16
NW = NC * NS
L = 16
BB = 128
HG = 8


def _emb_call(H, B, V2, D, n_hg):
    mesh = plsc.VectorSubcoreMesh(core_axis_name="c", subcore_axis_name="s")
    n_grp = BB // L

    @functools.partial(
        pl.kernel,
        mesh=mesh,
        out_type=jax.ShapeDtypeStruct((H, D, B), jnp.float32),
        scratch_types=[
            pltpu.VMEM((HG, BB), jnp.int32),
            pltpu.VMEM((HG, BB), jnp.int32),
            pltpu.VMEM((HG, BB), jnp.int32),
            pltpu.VMEM((BB, 2 * D), jnp.float32),
            pltpu.VMEM((D, BB), jnp.float32),
            pltpu.SemaphoreType.DMA,
            pltpu.SemaphoreType.DMA,
        ],
        compiler_params=pltpu.CompilerParams(
            use_tc_tiling_on_sc=True, needs_layout_passes=False),
    )
    def emb(xt_hbm, tab_hbm, out_hbm, idx_v, half_v, low_v, rows_v, slab_v,
            sem_g, sem_o):
        wid = lax.axis_index("s") * NC + lax.axis_index("c")
        b0 = wid * BB

        def unit(g, carry):
            pltpu.sync_copy(xt_hbm.at[pl.ds(g * HG, HG), pl.ds(b0, BB)], idx_v)
            def prep(r):
                for c in range(n_grp):
                    v = idx_v[r, pl.ds(c * L, L)]
                    half_v[r, pl.ds(c * L, L)] = lax.shift_right_logical(v, 1)
                    low_v[r, pl.ds(c * L, L)] = (v & 1) * D
            for r in range(HG):
                prep(r)

            def one_h(r):
                h = g * HG + r
                pltpu.async_copy(
                    tab_hbm.at[half_v.at[r]], rows_v, sem_g).wait()
                def dloop(d, carry2):
                    for c in range(n_grp):
                        bidx = lax.broadcasted_iota(jnp.int32, (L,), 0) + c * L
                        cidx = low_v[r, pl.ds(c * L, L)] + d
                        vals = plsc.load_gather(rows_v, [bidx, cidx])
                        slab_v[d, pl.ds(c * L, L)] = vals
                    return carry2
                lax.fori_loop(0, D, dloop, 0)
                pltpu.sync_copy(slab_v, out_hbm.at[h, :, pl.ds(b0, BB)])
            for r in range(HG):
                one_h(r)
            return carry

        lax.fori_loop(0, n_hg, unit, 0)

    return emb


def kernel(x, table):
    B, H = x.shape
    V, D = table.shape
    xt = x.T.astype(jnp.int32)
    tab2 = table.reshape(V // 2, 2 * D)
    n_hg = H // HG
    q = _emb_call(H, B, V // 2, D, n_hg)(xt, tab2)
    return q.transpose(2, 0, 1)


# speedup vs baseline: 1.8321x; 1.8321x over previous
import functools

import jax
import jax.numpy as jnp
from jax import lax
from jax.experimental import pallas as pl
from jax.experimental.pallas import tpu as pltpu
from jax.experimental.pallas import tpu_sc as plsc

NC = 2
NS = 16
NW = NC * NS
L = 16
BB = 128
HG = 8
NBUF = 2


def _emb_call(H, B, D, n_hg):
    mesh = plsc.VectorSubcoreMesh(core_axis_name="c", subcore_axis_name="s")
    n_grp = BB // L
    RW = 2 * D

    @functools.partial(
        pl.kernel,
        mesh=mesh,
        out_type=jax.ShapeDtypeStruct((H, D, B), jnp.float32),
        scratch_types=[
            pltpu.VMEM((HG, BB), jnp.int32),
            pltpu.VMEM((HG, BB), jnp.int32),
            pltpu.VMEM((HG, BB), jnp.int32),
            [pltpu.VMEM((BB, RW), jnp.float32) for _ in range(NBUF)],
            [pltpu.VMEM((D, BB), jnp.float32) for _ in range(NBUF)],
            [pltpu.SemaphoreType.DMA for _ in range(NBUF)],
            [pltpu.SemaphoreType.DMA for _ in range(NBUF)],
        ],
        compiler_params=pltpu.CompilerParams(
            use_tc_tiling_on_sc=True, needs_layout_passes=False),
    )
    def emb(xt_hbm, tab_hbm, out_hbm, idx_v, half_v, base_v, rows_v, slab_v,
            sem_g, sem_o):
        wid = lax.axis_index("s") * NC + lax.axis_index("c")
        b0 = wid * BB

        def gather_start(r, k):
            return pltpu.async_copy(
                tab_hbm.at[half_v.at[r]], rows_v[k], sem_g[k])

        def slab_copy(h, k):
            return pltpu.make_async_copy(
                slab_v[k],
                out_hbm.at[h, :, pl.ds(b0, BB)],
                sem_o[k],
            )

        def prep(g):
            pltpu.sync_copy(xt_hbm.at[pl.ds(g * HG, HG), pl.ds(b0, BB)],
                            idx_v)
            for r in range(HG):
                for c in range(n_grp):
                    s = pl.ds(c * L, L)
                    v = idx_v[r, s]
                    half_v[r, s] = lax.shift_right_logical(v, 1)
                    base_v[r, s] = (v & 1) * D

        lane = lax.broadcasted_iota(jnp.int32, (L,), 0)

        def transpose_h(r, k):
            for c in range(n_grp):
                bvec = lane + c * L
                colv = base_v[r, pl.ds(c * L, L)]

                @plsc.parallel_loop(0, D, 1, unroll=8)
                def _(d):
                    vals = plsc.load_gather(rows_v[k], [bvec, colv + d])
                    slab_v[k][d, pl.ds(c * L, L)] = vals

        def unit(g, carry):
            prep(g)
            for r in range(HG):
                h = g * HG + r
                k = r % NBUF
                @pl.when((g > 0) | (r >= NBUF))
                def _():
                    slab_copy(h - NBUF, k).wait()
                gather_start(r, k).wait()
                transpose_h(r, k)
                slab_copy(h, k).start()
            return carry

        lax.fori_loop(0, n_hg, unit, 0)
        for r in range(NBUF):
            slab_copy(H - NBUF + r, (H - NBUF + r) % NBUF).wait()

    return emb


def kernel(x, table):
    B, H = x.shape
    V, D = table.shape
    xt = x.T.astype(jnp.int32)
    tab2 = table.reshape(V // 2, 2 * D)
    n_hg = H // HG
    q = _emb_call(H, B, D, n_hg)(xt, tab2)
    return q.transpose(2, 0, 1)


# speedup vs baseline: 2.1016x; 1.1470x over previous
import functools

import jax
import jax.numpy as jnp
from jax import lax
from jax.experimental import pallas as pl
from jax.experimental.pallas import tpu as pltpu
from jax.experimental.pallas import tpu_sc as plsc

NC = 2
NS = 16
NW = NC * NS
L = 16
BB = 128
HG = 8
NBUF = 2
NG = 4


def _emb_call(H, B, D, n_hg):
    mesh = plsc.VectorSubcoreMesh(core_axis_name="c", subcore_axis_name="s")
    n_grp = BB // L
    RW = 2 * D

    @functools.partial(
        pl.kernel,
        mesh=mesh,
        out_type=jax.ShapeDtypeStruct((H, D, B), jnp.float32),
        scratch_types=[
            pltpu.VMEM((HG, BB), jnp.int32),
            pltpu.VMEM((HG, BB), jnp.int32),
            pltpu.VMEM((HG, BB), jnp.int32),
            [pltpu.VMEM((BB, RW), jnp.float32) for _ in range(NG)],
            [pltpu.VMEM((D, BB), jnp.float32) for _ in range(NBUF)],
            [pltpu.SemaphoreType.DMA for _ in range(NG)],
            [pltpu.SemaphoreType.DMA for _ in range(NBUF)],
        ],
        compiler_params=pltpu.CompilerParams(
            use_tc_tiling_on_sc=True, needs_layout_passes=False),
    )
    def emb(xt_hbm, tab_hbm, out_hbm, idx_v, half_v, base_v, rows_v, slab_v,
            sem_g, sem_o):
        wid = lax.axis_index("s") * NC + lax.axis_index("c")
        b0 = wid * BB
        lane = lax.broadcasted_iota(jnp.int32, (L,), 0)

        def gather_copy(r, k):
            return pltpu.make_async_copy(
                tab_hbm.at[half_v.at[r]], rows_v[k], sem_g[k])

        def slab_copy(h, k):
            return pltpu.make_async_copy(
                slab_v[k],
                out_hbm.at[h, :, pl.ds(b0, BB)],
                sem_o[k],
            )

        def prep(g):
            pltpu.sync_copy(xt_hbm.at[pl.ds(g * HG, HG), pl.ds(b0, BB)],
                            idx_v)
            for r in range(HG):
                for c in range(n_grp):
                    s = pl.ds(c * L, L)
                    v = idx_v[r, s]
                    half_v[r, s] = lax.shift_right_logical(v, 1)
                    base_v[r, s] = (v & 1) * D

        def transpose_h(r, kg, ks):
            for c in range(n_grp):
                bvec = lane + c * L
                colv = base_v[r, pl.ds(c * L, L)]

                @plsc.parallel_loop(0, D, 1, unroll=8)
                def _(d):
                    vals = plsc.load_gather(rows_v[kg], [bvec, colv + d])
                    slab_v[ks][d, pl.ds(c * L, L)] = vals

        def unit(g, carry):
            prep(g)
            for r in range(NG):
                gather_copy(r, r).start()
            for r in range(HG):
                h = g * HG + r
                kg = r % NG
                ks = r % NBUF
                @pl.when((g > 0) | (r >= NBUF))
                def _():
                    slab_copy(h - NBUF, ks).wait()
                gather_copy(r, kg).wait()
                transpose_h(r, kg, ks)
                slab_copy(h, ks).start()
                if r + NG < HG:
                    gather_copy(r + NG, kg).start()
            return carry

        lax.fori_loop(0, n_hg, unit, 0)
        for r in range(NBUF):
            slab_copy(H - NBUF + r, (H - NBUF + r) % NBUF).wait()

    return emb


def kernel(x, table):
    B, H = x.shape
    V, D = table.shape
    xt = x.T.astype(jnp.int32)
    tab2 = table.reshape(V // 2, 2 * D)
    n_hg = H // HG
    q = _emb_call(H, B, D, n_hg)(xt, tab2)
    return q.transpose(2, 0, 1)


# speedup vs baseline: 2.1257x; 1.0115x over previous
import functools

import jax
import jax.numpy as jnp
from jax import lax
from jax.experimental import pallas as pl
from jax.experimental.pallas import tpu as pltpu
from jax.experimental.pallas import tpu_sc as plsc

NC = 2
NS = 16
NW = NC * NS
L = 16
BB = 128
HG = 8
NBUF = 2
NG = 4


def _emb_call(H, B, D, n_hg):
    mesh = plsc.VectorSubcoreMesh(core_axis_name="c", subcore_axis_name="s")
    n_grp = BB // L
    RW = 2 * D

    @functools.partial(
        pl.kernel,
        mesh=mesh,
        out_type=jax.ShapeDtypeStruct((H, D, B), jnp.float32),
        scratch_types=[
            pltpu.VMEM((HG, BB), jnp.int32),
            pltpu.VMEM((HG, BB), jnp.int32),
            pltpu.VMEM((HG, BB), jnp.int32),
            [pltpu.VMEM((BB, RW), jnp.float32) for _ in range(NG)],
            [pltpu.VMEM((D, BB), jnp.float32) for _ in range(NBUF)],
            [pltpu.SemaphoreType.DMA for _ in range(NG)],
            [pltpu.SemaphoreType.DMA for _ in range(NBUF)],
        ],
        compiler_params=pltpu.CompilerParams(
            use_tc_tiling_on_sc=True, needs_layout_passes=False),
    )
    def emb(xt_hbm, tab_hbm, out_hbm, idx_v, half_v, base_v, rows_v, slab_v,
            sem_g, sem_o):
        wid = lax.axis_index("s") * NC + lax.axis_index("c")
        b0 = wid * BB
        lane = lax.broadcasted_iota(jnp.int32, (L,), 0)

        def gather_copy(r, k):
            return pltpu.make_async_copy(
                tab_hbm.at[half_v.at[r]], rows_v[k], sem_g[k])

        def slab_copy(h, k):
            return pltpu.make_async_copy(
                slab_v[k],
                out_hbm.at[h, :, pl.ds(b0, BB)],
                sem_o[k],
            )

        def prep(g):
            pltpu.sync_copy(xt_hbm.at[pl.ds(g * HG, HG), pl.ds(b0, BB)],
                            idx_v)
            for r in range(HG):
                for c in range(n_grp):
                    s = pl.ds(c * L, L)
                    v = idx_v[r, s]
                    half_v[r, s] = lax.shift_right_logical(v, 1)
                    base_v[r, s] = (lane + c * L) * RW + (v & 1) * D

        zerov = lax.broadcasted_iota(jnp.int32, (L,), 0) * 0

        def transpose_h(r, kg, ks):
            cols0 = tuple(
                base_v[r, pl.ds(c * L, L)] for c in range(n_grp))

            @plsc.parallel_loop(0, D, 1, unroll=4, carry=cols0)
            def _(d, cols):
                for c in range(n_grp):
                    vals = plsc.load_gather(rows_v[kg], [zerov, cols[c]])
                    slab_v[ks][d, pl.ds(c * L, L)] = vals
                return tuple(col + 1 for col in cols)

        def unit(g, carry):
            prep(g)
            for r in range(NG):
                gather_copy(r, r).start()
            for r in range(HG):
                h = g * HG + r
                kg = r % NG
                ks = r % NBUF
                @pl.when((g > 0) | (r >= NBUF))
                def _():
                    slab_copy(h - NBUF, ks).wait()
                gather_copy(r, kg).wait()
                transpose_h(r, kg, ks)
                slab_copy(h, ks).start()
                if r + NG < HG:
                    gather_copy(r + NG, kg).start()
            return carry

        lax.fori_loop(0, n_hg, unit, 0)
        for r in range(NBUF):
            slab_copy(H - NBUF + r, (H - NBUF + r) % NBUF).wait()

    return emb


def kernel(x, table):
    B, H = x.shape
    V, D = table.shape
    xt = x.T.astype(jnp.int32)
    tab2 = table.reshape(V // 2, 2 * D)
    n_hg = H // HG
    q = _emb_call(H, B, D, n_hg)(xt, tab2)
    return q.transpose(2, 0, 1)


# speedup vs baseline: 2.1267x; 1.0005x over previous
import functools

import jax
import jax.numpy as jnp
from jax import lax
from jax.experimental import pallas as pl
from jax.experimental.pallas import tpu as pltpu
from jax.experimental.pallas import tpu_sc as plsc

NC = 2
NS = 16
NW = NC * NS
L = 16
BB = 128
HG = 8
NBUF = 2
NG = 4


def _emb_call(H, B, D, n_hg):
    mesh = plsc.VectorSubcoreMesh(core_axis_name="c", subcore_axis_name="s")
    n_grp = BB // L
    RW = 2 * D

    @functools.partial(
        pl.kernel,
        mesh=mesh,
        out_type=jax.ShapeDtypeStruct((H, D, B), jnp.float32),
        scratch_types=[
            pltpu.VMEM((HG, BB), jnp.int32),
            pltpu.VMEM((HG, BB), jnp.int32),
            pltpu.VMEM((HG, BB), jnp.int32),
            [pltpu.VMEM((BB, RW), jnp.float32) for _ in range(NG)],
            [pltpu.VMEM((D, BB), jnp.float32) for _ in range(NBUF)],
            [pltpu.SemaphoreType.DMA for _ in range(NG)],
            [pltpu.SemaphoreType.DMA for _ in range(NBUF)],
        ],
        compiler_params=pltpu.CompilerParams(
            use_tc_tiling_on_sc=True, needs_layout_passes=False),
    )
    def emb(xt_hbm, tab_hbm, out_hbm, idx_v, half_v, base_v, rows_v, slab_v,
            sem_g, sem_o):
        wid = lax.axis_index("s") * NC + lax.axis_index("c")
        b0 = wid * BB
        lane = lax.broadcasted_iota(jnp.int32, (L,), 0)

        def gather_copy(r, k):
            return pltpu.make_async_copy(
                tab_hbm.at[half_v.at[r]], rows_v[k], sem_g[k])

        def slab_copy(h, k):
            return pltpu.make_async_copy(
                slab_v[k],
                out_hbm.at[h, :, pl.ds(b0, BB)],
                sem_o[k],
            )

        def prep(g):
            pltpu.sync_copy(xt_hbm.at[pl.ds(g * HG, HG), pl.ds(b0, BB)],
                            idx_v)
            for r in range(HG):
                for c in range(n_grp):
                    s = pl.ds(c * L, L)
                    v = idx_v[r, s]
                    half_v[r, s] = lax.shift_right_logical(v, 1)
                    base_v[r, s] = (lane + c * L) * RW + (v & 1) * D

        zerov = lax.broadcasted_iota(jnp.int32, (L,), 0) * 0

        def transpose_h(r, kg, ks):
            cols0 = tuple(
                base_v[r, pl.ds(c * L, L)] for c in range(n_grp))

            @plsc.parallel_loop(0, D, 1, unroll=8)
            def _(d):
                for c in range(n_grp):
                    vals = plsc.load_gather(rows_v[kg], [zerov, cols0[c] + d])
                    slab_v[ks][d, pl.ds(c * L, L)] = vals

        def unit(g, carry):
            prep(g)
            for r in range(NG):
                gather_copy(r, r).start()
            for r in range(HG):
                h = g * HG + r
                kg = r % NG
                ks = r % NBUF
                @pl.when((g > 0) | (r >= NBUF))
                def _():
                    slab_copy(h - NBUF, ks).wait()
                gather_copy(r, kg).wait()
                transpose_h(r, kg, ks)
                slab_copy(h, ks).start()
                if r + NG < HG:
                    gather_copy(r + NG, kg).start()
            return carry

        lax.fori_loop(0, n_hg, unit, 0)
        for r in range(NBUF):
            slab_copy(H - NBUF + r, (H - NBUF + r) % NBUF).wait()

    return emb


def kernel(x, table):
    B, H = x.shape
    V, D = table.shape
    xt = x.T.astype(jnp.int32)
    tab2 = table.reshape(V // 2, 2 * D)
    n_hg = H // HG
    q = _emb_call(H, B, D, n_hg)(xt, tab2)
    return q.transpose(2, 0, 1)
